# Initial kernel scaffold; baseline (speedup 1.0000x reference)
import jax
import jax.numpy as jnp
from jax.experimental import pallas as pl


def kernel(xyz, num_class, skel_xyz):
    raise NotImplementedError("write your pallas kernel here")


# speedup vs baseline: 12.6105x; 12.6105x over previous
import functools

import jax
import jax.numpy as jnp
from jax.experimental import pallas as pl

B = 4
N = 4096
K = 15
R = 256


def _loss_block(pts_ref, ptsT_ref, nrm_ref, nrmT_ref, out_ref):
    p = pts_ref[0]
    q = ptsT_ref[0]
    d2 = (
        jnp.sum(p * p, axis=1, keepdims=True)
        + jnp.sum(q * q, axis=0, keepdims=True)
        - 2.0 * jnp.dot(p, q, preferred_element_type=jnp.float32)
    )

    n = nrm_ref[0]
    m = nrmT_ref[0]
    nx, ny, nz = n[:, 0:1], n[:, 1:2], n[:, 2:3]
    mx, my, mz = m[0:1, :], m[1:2, :], m[2:3, :]
    cx = ny * mz - nz * my
    cy = nz * mx - nx * mz
    cz = nx * my - ny * mx
    cross2 = cx * cx + cy * cy + cz * cz
    px, py, pz = nx * mx, ny * my, nz * mz
    sq = px * px + py * py + pz * pz
    f = jnp.sqrt(jnp.minimum(cross2, sq))

    acc = jnp.zeros((R, 1), jnp.float32)
    cnt = jnp.zeros((R, 1), jnp.float32)
    d2w = d2
    for _ in range(K):
        mn = jnp.min(d2w, axis=1, keepdims=True)
        eq = d2w == mn
        nt = jnp.sum(eq.astype(jnp.float32), axis=1, keepdims=True)
        sf = jnp.sum(jnp.where(eq, f, 0.0), axis=1, keepdims=True)
        take = jnp.clip(float(K) - cnt, 0.0, nt)
        acc = acc + sf * take / nt
        cnt = cnt + take
        d2w = jnp.where(eq, jnp.inf, d2w)

    out_ref[...] = acc.reshape(1, 1, 1, R)


@jax.jit
def _loss(xyz):
    pts = xyz[:, :, 0:3]
    nrm = xyz[:, :, 3:6]
    ptsT = pts.transpose(0, 2, 1)
    nrmT = nrm.transpose(0, 2, 1)
    nb = N // R
    out = pl.pallas_call(
        _loss_block,
        grid=(B, nb),
        in_specs=[
            pl.BlockSpec((1, R, 3), lambda b, rb: (b, rb, 0)),
            pl.BlockSpec((1, 3, N), lambda b, rb: (b, 0, 0)),
            pl.BlockSpec((1, R, 3), lambda b, rb: (b, rb, 0)),
            pl.BlockSpec((1, 3, N), lambda b, rb: (b, 0, 0)),
        ],
        out_specs=pl.BlockSpec((1, 1, 1, R), lambda b, rb: (b, rb, 0, 0)),
        out_shape=jax.ShapeDtypeStruct((B, nb, 1, R), jnp.float32),
    )(pts, ptsT, nrm, nrmT)
    mean = jnp.sum(out) / float(B * N)
    return 1.0 * mean + 1.5 * mean


def kernel(xyz, num_class, skel_xyz):
    del num_class, skel_xyz
    return _loss(xyz)


# speedup vs baseline: 21.8753x; 1.7347x over previous
import functools

import jax
import jax.numpy as jnp
from jax.experimental import pallas as pl

B = 4
N = 4096
K = 15
R = 256


def _loss_block(pts_ref, ptsT_ref, nrm_ref, nrmT_ref, out_ref):
    p = pts_ref[0]
    q = ptsT_ref[0]
    d2 = (
        jnp.sum(p * p, axis=1, keepdims=True)
        + jnp.sum(q * q, axis=0, keepdims=True)
        - 2.0 * jnp.dot(p, q, preferred_element_type=jnp.float32)
    )

    d2w = d2
    mn = jnp.min(d2w, axis=1, keepdims=True)
    for _ in range(K - 1):
        d2w = jnp.where(d2w == mn, jnp.inf, d2w)
        mn = jnp.min(d2w, axis=1, keepdims=True)
    t = mn

    n = nrm_ref[0]
    m = nrmT_ref[0]
    nx, ny, nz = n[:, 0:1], n[:, 1:2], n[:, 2:3]
    mx, my, mz = m[0:1, :], m[1:2, :], m[2:3, :]
    cx = ny * mz - nz * my
    cy = nz * mx - nx * mz
    cz = nx * my - ny * mx
    cross2 = cx * cx + cy * cy + cz * cz
    px, py, pz = nx * mx, ny * my, nz * mz
    sq = px * px + py * py + pz * pz
    f = jnp.sqrt(jnp.minimum(cross2, sq))

    lt = d2 < t
    eqm = d2 == t
    sf_lt = jnp.sum(jnp.where(lt, f, 0.0), axis=1, keepdims=True)
    clt = jnp.sum(lt.astype(jnp.float32), axis=1, keepdims=True)
    sf_eq = jnp.sum(jnp.where(eqm, f, 0.0), axis=1, keepdims=True)
    ne = jnp.sum(eqm.astype(jnp.float32), axis=1, keepdims=True)
    acc = sf_lt + sf_eq * jnp.clip(float(K) - clt, 0.0, ne) / jnp.maximum(ne, 1.0)

    out_ref[...] = acc.reshape(1, 1, 1, R)


@jax.jit
def _loss(xyz):
    pts = xyz[:, :, 0:3]
    nrm = xyz[:, :, 3:6]
    ptsT = pts.transpose(0, 2, 1)
    nrmT = nrm.transpose(0, 2, 1)
    nb = N // R
    out = pl.pallas_call(
        _loss_block,
        grid=(B, nb),
        in_specs=[
            pl.BlockSpec((1, R, 3), lambda b, rb: (b, rb, 0)),
            pl.BlockSpec((1, 3, N), lambda b, rb: (b, 0, 0)),
            pl.BlockSpec((1, R, 3), lambda b, rb: (b, rb, 0)),
            pl.BlockSpec((1, 3, N), lambda b, rb: (b, 0, 0)),
        ],
        out_specs=pl.BlockSpec((1, 1, 1, R), lambda b, rb: (b, rb, 0, 0)),
        out_shape=jax.ShapeDtypeStruct((B, nb, 1, R), jnp.float32),
    )(pts, ptsT, nrm, nrmT)
    mean = jnp.sum(out) / float(B * N)
    return 1.0 * mean + 1.5 * mean


def kernel(xyz, num_class, skel_xyz):
    del num_class, skel_xyz
    return _loss(xyz)


# speedup vs baseline: 24.7790x; 1.1327x over previous
import functools

import jax
import jax.numpy as jnp
from jax.experimental import pallas as pl
from jax.experimental.pallas import tpu as pltpu

B = 4
N = 4096
K = 15
R = 256


def _loss_block(pts_ref, ptsT_ref, nrm_ref, nrmT_ref, out_ref):
    p = pts_ref[0]
    q = ptsT_ref[0]
    d2 = (
        jnp.sum(p * p, axis=1, keepdims=True)
        + jnp.sum(q * q, axis=0, keepdims=True)
        - 2.0 * jnp.dot(p, q, preferred_element_type=jnp.float32)
    )

    d2w = d2
    mn = jnp.min(d2w, axis=1, keepdims=True)
    for _ in range(K - 1):
        d2w = jnp.where(d2w == mn, jnp.inf, d2w)
        mn = jnp.min(d2w, axis=1, keepdims=True)
    t = mn

    n = nrm_ref[0]
    m = nrmT_ref[0]
    nx, ny, nz = n[:, 0:1], n[:, 1:2], n[:, 2:3]
    mx, my, mz = m[0:1, :], m[1:2, :], m[2:3, :]
    px, py, pz = nx * mx, ny * my, nz * mz
    dot = px + py + pz
    sq = px * px + py * py + pz * pz
    nn2 = nx * nx + ny * ny + nz * nz
    mm2 = mx * mx + my * my + mz * mz
    cross2 = jnp.maximum(nn2 * mm2 - dot * dot, 0.0)
    f = jnp.sqrt(jnp.minimum(cross2, sq))

    lt = d2 < t
    eqm = d2 == t
    sf_lt = jnp.sum(jnp.where(lt, f, 0.0), axis=1, keepdims=True)
    clt = jnp.sum(lt.astype(jnp.float32), axis=1, keepdims=True)
    sf_eq = jnp.sum(jnp.where(eqm, f, 0.0), axis=1, keepdims=True)
    ne = jnp.sum(eqm.astype(jnp.float32), axis=1, keepdims=True)
    acc = sf_lt + sf_eq * jnp.clip(float(K) - clt, 0.0, ne) / jnp.maximum(ne, 1.0)

    out_ref[...] = acc.reshape(1, 1, 1, R)


@jax.jit
def _loss(xyz):
    pts = xyz[:, :, 0:3]
    nrm = xyz[:, :, 3:6]
    ptsT = pts.transpose(0, 2, 1)
    nrmT = nrm.transpose(0, 2, 1)
    nb = N // R
    out = pl.pallas_call(
        _loss_block,
        grid=(B, nb),
        in_specs=[
            pl.BlockSpec((1, R, 3), lambda b, rb: (b, rb, 0)),
            pl.BlockSpec((1, 3, N), lambda b, rb: (b, 0, 0)),
            pl.BlockSpec((1, R, 3), lambda b, rb: (b, rb, 0)),
            pl.BlockSpec((1, 3, N), lambda b, rb: (b, 0, 0)),
        ],
        out_specs=pl.BlockSpec((1, 1, 1, R), lambda b, rb: (b, rb, 0, 0)),
        out_shape=jax.ShapeDtypeStruct((B, nb, 1, R), jnp.float32),
        compiler_params=pltpu.CompilerParams(
            dimension_semantics=("parallel", "parallel")),
    )(pts, ptsT, nrm, nrmT)
    mean = jnp.sum(out) / float(B * N)
    return 1.0 * mean + 1.5 * mean


def kernel(xyz, num_class, skel_xyz):
    del num_class, skel_xyz
    return _loss(xyz)
